# Initial kernel scaffold; baseline (speedup 1.0000x reference)
import jax
import jax.numpy as jnp
from jax.experimental import pallas as pl


def kernel(input_tensor, spa_cate, spa_flag, spa_temp_flag, spa_temp, W_cate, W_flag, W_tflag):
    raise NotImplementedError("write your pallas kernel here")


# speedup vs baseline: 16.1112x; 16.1112x over previous
import dataclasses
import functools

import jax
import jax.numpy as jnp
from jax import lax
from jax.experimental import pallas as pl
from jax.experimental.pallas import tpu as pltpu
from jax.experimental.pallas import tpu_sc as plsc

N, T, C, L = 16, 12, 16, 1024
D_CATE, N_CATE = 16, 3
N_FLAG, D_FLAG = 4, 4
N_TFLAG, D_TFLAG = 9, 4
C_OUT = C + N_CATE * D_CATE + N_FLAG * D_FLAG + N_TFLAG * D_TFLAG + 6

NC, NS = 2, 16
NW = NC * NS
B_GATHER = N * N_CATE * L
PER_W = B_GATHER // NW
CHUNK = 128
NCHUNK = PER_W // CHUNK


GRP = 8


def _sc_gather(table3, idx3):
    V = table3.shape[0] * GRP // N_CATE
    mesh = plsc.VectorSubcoreMesh(core_axis_name="c", subcore_axis_name="s")
    cp = pltpu.CompilerParams()
    if "needs_layout_passes" in pltpu.CompilerParams.__dataclass_fields__:
        cp = dataclasses.replace(cp, needs_layout_passes=False)

    @functools.partial(
        pl.kernel,
        out_type=jax.ShapeDtypeStruct((B_GATHER * D_CATE,), jnp.float32),
        mesh=mesh,
        compiler_params=cp,
        scratch_types=[
            pltpu.VMEM((NCHUNK, CHUNK), jnp.int32),
            pltpu.VMEM((NCHUNK, CHUNK), jnp.int32),
            pltpu.VMEM((CHUNK, GRP * D_CATE), jnp.float32),
            pltpu.VMEM((PER_W * D_CATE,), jnp.float32),
            pltpu.SemaphoreType.DMA,
        ],
    )
    def k(table_hbm, idx_hbm, out_hbm, idx_v, sub_v, gbuf, rows_v, sem):
        wid = lax.axis_index("s") * NC + lax.axis_index("c")
        base = wid * PER_W
        pltpu.sync_copy(idx_hbm.at[wid], idx_v)
        for j in range(NCHUNK):
            toff = ((base + j * CHUNK) // L) % N_CATE * V
            for c in range(CHUNK // 16):
                sl = (j, pl.ds(c * 16, 16))
                e = idx_v[sl] + toff
                sub_v[sl] = jnp.bitwise_and(e, GRP - 1)
                idx_v[sl] = jnp.right_shift(e, 3)

        iota16 = lax.iota(jnp.int32, 16)

        @pl.loop(0, NCHUNK)
        def _(j):
            pltpu.async_copy(table_hbm.at[idx_v.at[j]], gbuf, sem).wait()
            for g in range(CHUNK // 16):
                kvec = iota16 + g * 16
                sub16 = sub_v[j, pl.ds(g * 16, 16)]
                obase = (j * CHUNK + kvec) * D_CATE
                col0 = sub16 * D_CATE
                for d in range(D_CATE):
                    v = plsc.load_gather(gbuf, [kvec, col0 + d])
                    plsc.store_scatter(rows_v, [obase + d], v)

        pltpu.sync_copy(rows_v, out_hbm.at[pl.ds(base * D_CATE, PER_W * D_CATE)])

    return k(table3, idx3)


def _tc_assemble(input_tensor, rows4, flag_f, tflag_f, spa_temp, A, w0, Bm, tw0):
    C_SC = N_CATE * D_CATE
    C_SF = N_FLAG * D_FLAG
    C_TF = N_TFLAG * D_TFLAG

    def body(inp, rows, flag, tflag, temp, a, w0r, bm, tw0r, out, sc_s):
        t = pl.program_id(1)

        @pl.when(t == 0)
        def _():
            for i in range(N_CATE):
                sc_s[i * D_CATE:(i + 1) * D_CATE, :] = rows[0, i].T

        sf = jnp.dot(a[...], flag[0], preferred_element_type=jnp.float32) + w0r[...]
        stf = jnp.dot(bm[...], tflag[0, 0], preferred_element_type=jnp.float32) + tw0r[...]
        out[0, 0] = jnp.concatenate(
            [inp[0, 0], sc_s[...], sf, stf, temp[0, 0]], axis=0)

    return pl.pallas_call(
        body,
        grid=(N, T),
        in_specs=[
            pl.BlockSpec((1, 1, C, L), lambda n, t: (n, t, 0, 0)),
            pl.BlockSpec((1, N_CATE, L, D_CATE), lambda n, t: (n, 0, 0, 0)),
            pl.BlockSpec((1, N_FLAG, L), lambda n, t: (n, 0, 0)),
            pl.BlockSpec((1, 1, N_TFLAG, L), lambda n, t: (n, t, 0, 0)),
            pl.BlockSpec((1, 1, 6, L), lambda n, t: (n, t, 0, 0)),
            pl.BlockSpec((C_SF, N_FLAG), lambda n, t: (0, 0)),
            pl.BlockSpec((C_SF, 1), lambda n, t: (0, 0)),
            pl.BlockSpec((C_TF, N_TFLAG), lambda n, t: (0, 0)),
            pl.BlockSpec((C_TF, 1), lambda n, t: (0, 0)),
        ],
        out_specs=pl.BlockSpec((1, 1, C_OUT, L), lambda n, t: (n, t, 0, 0)),
        out_shape=jax.ShapeDtypeStruct((N, T, C_OUT, L), jnp.float32),
        scratch_shapes=[pltpu.VMEM((C_SC, L), jnp.float32)],
    )(input_tensor, rows4, flag_f, tflag_f, spa_temp, A, w0, Bm, tw0)


def kernel(input_tensor, spa_cate, spa_flag, spa_temp_flag, spa_temp,
           W_cate, W_flag, W_tflag):
    V = W_cate.shape[1]
    table3 = W_cate.reshape(N_CATE * V // GRP, GRP * D_CATE)
    idx3 = spa_cate.astype(jnp.int32).reshape(NW, NCHUNK, CHUNK)
    rows = _sc_gather(table3, idx3)
    rows4 = rows.reshape(N, N_CATE, L, D_CATE)

    wd = W_flag[:, 1, :] - W_flag[:, 0, :]
    A = (jnp.eye(N_FLAG, dtype=jnp.float32)[:, None, :]
         * wd[:, :, None]).reshape(N_FLAG * D_FLAG, N_FLAG)
    w0 = W_flag[:, 0, :].reshape(N_FLAG * D_FLAG, 1)
    wtd = W_tflag[:, 1, :] - W_tflag[:, 0, :]
    Bm = (jnp.eye(N_TFLAG, dtype=jnp.float32)[:, None, :]
          * wtd[:, :, None]).reshape(N_TFLAG * D_TFLAG, N_TFLAG)
    tw0 = W_tflag[:, 0, :].reshape(N_TFLAG * D_TFLAG, 1)

    flag_f = spa_flag.astype(jnp.float32)
    tflag_f = spa_temp_flag.astype(jnp.float32)

    return _tc_assemble(input_tensor, rows4, flag_f, tflag_f, spa_temp,
                        A, w0, Bm, tw0)


# speedup vs baseline: 23.0651x; 1.4316x over previous
import dataclasses
import functools

import jax
import jax.numpy as jnp
from jax import lax
from jax.experimental import pallas as pl
from jax.experimental.pallas import tpu as pltpu
from jax.experimental.pallas import tpu_sc as plsc

N, T, C, L = 16, 12, 16, 1024
D_CATE, N_CATE = 16, 3
N_FLAG, D_FLAG = 4, 4
N_TFLAG, D_TFLAG = 9, 4
C_OUT = C + N_CATE * D_CATE + N_FLAG * D_FLAG + N_TFLAG * D_TFLAG + 6
C_SC = N_CATE * D_CATE
C_SF = N_FLAG * D_FLAG
C_TF = N_TFLAG * D_TFLAG

NC, NS = 2, 16
NW = NC * NS
B_GATHER = N * N_CATE * L
PER_W = B_GATHER // NW
CHUNK = 128
NCHUNK = PER_W // CHUNK
GRP = 8
HSEG = 512
NHSEG = PER_W // HSEG
CPH = HSEG // CHUNK


def _sc_gather(table2, idx3):
    V = table2.shape[0] * GRP // N_CATE
    mesh = plsc.VectorSubcoreMesh(core_axis_name="c", subcore_axis_name="s")
    cp = pltpu.CompilerParams()
    if "needs_layout_passes" in pltpu.CompilerParams.__dataclass_fields__:
        cp = dataclasses.replace(cp, needs_layout_passes=False)

    @functools.partial(
        pl.kernel,
        out_type=jax.ShapeDtypeStruct((N, C_SC, L), jnp.float32),
        mesh=mesh,
        compiler_params=cp,
        scratch_types=[
            pltpu.VMEM((NCHUNK, CHUNK), jnp.int32),
            pltpu.VMEM((NCHUNK, CHUNK), jnp.int32),
            pltpu.VMEM((CHUNK, GRP * D_CATE), jnp.float32),
            pltpu.VMEM((CHUNK, GRP * D_CATE), jnp.float32),
            pltpu.VMEM((NHSEG, D_CATE, HSEG), jnp.float32),
            pltpu.SemaphoreType.DMA,
            pltpu.SemaphoreType.DMA,
        ],
    )
    def k(table_hbm, idx_hbm, out_hbm, idx_v, sub_v, gbuf0, gbuf1,
          rows_t, sem0, sem1):
        wid = lax.axis_index("s") * NC + lax.axis_index("c")
        base = wid * PER_W
        pltpu.sync_copy(idx_hbm.at[wid], idx_v)
        for j in range(NCHUNK):
            toff = ((base + j * CHUNK) // L) % N_CATE * V
            for c in range(CHUNK // 16):
                sl = (j, pl.ds(c * 16, 16))
                e = idx_v[sl] + toff
                sub_v[sl] = jnp.bitwise_and(e, GRP - 1)
                idx_v[sl] = jnp.right_shift(e, 3)

        iota16 = lax.iota(jnp.int32, 16)
        gbufs = (gbuf0, gbuf1)
        sems = (sem0, sem1)

        def fire(j, slot):
            return pltpu.async_copy(table_hbm.at[idx_v.at[j]],
                                    gbufs[slot], sems[slot])

        def extract(j, slot, hh, jj):
            gbuf = gbufs[slot]
            for g in range(CHUNK // 16):
                sub16 = sub_v[j, pl.ds(g * 16, 16)]
                kvec = iota16 + g * 16
                col0 = sub16 * D_CATE
                for d in range(D_CATE):
                    v = plsc.load_gather(gbuf, [kvec, col0 + d])
                    rows_t[hh, d, pl.ds(jj * CHUNK + g * 16, 16)] = v

        pend = fire(0, 0)
        for j in range(NCHUNK):
            slot = j & 1
            pend.wait()
            if j + 1 < NCHUNK:
                pend = fire(j + 1, 1 - slot)
            extract(j, slot, j // CPH, j % CPH)

        for hh in range(NHSEG):
            h = wid * NHSEG + hh
            n_h = h // (2 * N_CATE)
            i_h = (h // 2) % N_CATE
            l_h = (h % 2) * HSEG
            pltpu.sync_copy(
                rows_t.at[hh],
                out_hbm.at[n_h, pl.ds(i_h * D_CATE, D_CATE), pl.ds(l_h, HSEG)],
            )

    return k(table2, idx3)


def _tc_assemble(input_tensor, sc_rows, spa_flag, spa_temp_flag, spa_temp,
                 A, w0, Bm, tw0):
    def body(inp, rows, flag, tflag, temp, a, w0r, bm, tw0r, out):
        scb = rows[0]
        sf = jnp.dot(a[...], flag[0].astype(jnp.float32),
                     preferred_element_type=jnp.float32) + w0r[...]
        for t in range(T):
            stf = jnp.dot(bm[...], tflag[0, t].astype(jnp.float32),
                          preferred_element_type=jnp.float32) + tw0r[...]
            out[0, t] = jnp.concatenate(
                [inp[0, t], scb, sf, stf, temp[0, t]], axis=0)

    return pl.pallas_call(
        body,
        grid=(N,),
        in_specs=[
            pl.BlockSpec((1, T, C, L), lambda n: (n, 0, 0, 0)),
            pl.BlockSpec((1, C_SC, L), lambda n: (n, 0, 0)),
            pl.BlockSpec((1, N_FLAG, L), lambda n: (n, 0, 0)),
            pl.BlockSpec((1, T, N_TFLAG, L), lambda n: (n, 0, 0, 0)),
            pl.BlockSpec((1, T, 6, L), lambda n: (n, 0, 0, 0)),
            pl.BlockSpec((C_SF, N_FLAG), lambda n: (0, 0)),
            pl.BlockSpec((C_SF, 1), lambda n: (0, 0)),
            pl.BlockSpec((C_TF, N_TFLAG), lambda n: (0, 0)),
            pl.BlockSpec((C_TF, 1), lambda n: (0, 0)),
        ],
        out_specs=pl.BlockSpec((1, T, C_OUT, L), lambda n: (n, 0, 0, 0)),
        out_shape=jax.ShapeDtypeStruct((N, T, C_OUT, L), jnp.float32),
    )(input_tensor, sc_rows, spa_flag, spa_temp_flag, spa_temp,
      A, w0, Bm, tw0)


def kernel(input_tensor, spa_cate, spa_flag, spa_temp_flag, spa_temp,
           W_cate, W_flag, W_tflag):
    V = W_cate.shape[1]
    table2 = W_cate.reshape(N_CATE * V // GRP, GRP * D_CATE)
    idx3 = spa_cate.astype(jnp.int32).reshape(NW, NCHUNK, CHUNK)
    sc_rows = _sc_gather(table2, idx3)

    wd = W_flag[:, 1, :] - W_flag[:, 0, :]
    A = (jnp.eye(N_FLAG, dtype=jnp.float32)[:, None, :]
         * wd[:, :, None]).reshape(C_SF, N_FLAG)
    w0 = W_flag[:, 0, :].reshape(C_SF, 1)
    wtd = W_tflag[:, 1, :] - W_tflag[:, 0, :]
    Bm = (jnp.eye(N_TFLAG, dtype=jnp.float32)[:, None, :]
          * wtd[:, :, None]).reshape(C_TF, N_TFLAG)
    tw0 = W_tflag[:, 0, :].reshape(C_TF, 1)

    return _tc_assemble(input_tensor, sc_rows, spa_flag, spa_temp_flag,
                        spa_temp, A, w0, Bm, tw0)


# speedup vs baseline: 69.7305x; 3.0232x over previous
import dataclasses
import functools

import jax
import jax.numpy as jnp
from jax import lax
from jax.experimental import pallas as pl
from jax.experimental.pallas import tpu as pltpu
from jax.experimental.pallas import tpu_sc as plsc

N, T, C, L = 16, 12, 16, 1024
D_CATE, N_CATE = 16, 3
N_FLAG, D_FLAG = 4, 4
N_TFLAG, D_TFLAG = 9, 4
C_OUT = C + N_CATE * D_CATE + N_FLAG * D_FLAG + N_TFLAG * D_TFLAG + 6
C_SC = N_CATE * D_CATE
C_SF = N_FLAG * D_FLAG
C_TF = N_TFLAG * D_TFLAG

NC, NS = 2, 16
NW = NC * NS
TASKS = (C_SC * 2) // NW
NHALF = N // 2


def _sc_gather(wt, sct):
    V = wt.shape[2]
    mesh = plsc.VectorSubcoreMesh(core_axis_name="c", subcore_axis_name="s")
    cp = pltpu.CompilerParams()
    if "needs_layout_passes" in pltpu.CompilerParams.__dataclass_fields__:
        cp = dataclasses.replace(cp, needs_layout_passes=False)

    @functools.partial(
        pl.kernel,
        out_type=jax.ShapeDtypeStruct((C_SC, N, L), jnp.float32),
        mesh=mesh,
        compiler_params=cp,
        scratch_types=[
            pltpu.VMEM((V,), jnp.float32),
            pltpu.VMEM((NHALF, L), jnp.int32),
            pltpu.VMEM((NHALF, L), jnp.float32),
        ],
    )
    def k(wt_hbm, idx_hbm, out_hbm, row_v, idx_v, out_v):
        wid = lax.axis_index("s") * NC + lax.axis_index("c")
        h0 = wid * TASKS
        for kk in range(TASKS):
            h = h0 + kk
            pair = h // 2
            ti = pair // D_CATE
            td = pair % D_CATE
            nh = (h % 2) * NHALF
            if kk == 0:
                pltpu.sync_copy(wt_hbm.at[ti, td], row_v)
            else:
                @pl.when(pair != (h - 1) // 2)
                def _():
                    pltpu.sync_copy(wt_hbm.at[ti, td], row_v)
            pltpu.sync_copy(idx_hbm.at[ti, pl.ds(nh, NHALF)], idx_v)

            @pl.loop(0, NHALF)
            def _(nn):
                @pl.loop(0, L // 16)
                def _(g):
                    sl = pl.ds(g * 16, 16)
                    out_v[nn, sl] = plsc.load_gather(row_v, [idx_v[nn, sl]])

            pltpu.sync_copy(out_v, out_hbm.at[pair, pl.ds(nh, NHALF)])

    return k(wt, sct)


def _tc_assemble(inp_t, sc_rows, flag_t, tflag_t, temp_t, wd, w0, wtd, tw0):
    def body(inp, rows, flag, tflag, temp, wdr, w0r, wtdr, tw0r, out):
        fl = flag[...].astype(jnp.float32)
        tf = tflag[0].astype(jnp.float32)
        sf = [w0r[c] + wdr[c] * fl[c // D_FLAG] for c in range(C_SF)]
        stf = [tw0r[c] + wtdr[c] * tf[c // D_TFLAG] for c in range(C_TF)]
        out[0] = jnp.concatenate(
            [inp[0], rows[...], jnp.stack(sf), jnp.stack(stf), temp[0]],
            axis=0)

    smem = functools.partial(pl.BlockSpec, memory_space=pltpu.SMEM)
    return pl.pallas_call(
        body,
        grid=(T,),
        in_specs=[
            pl.BlockSpec((1, C, N, L), lambda t: (t, 0, 0, 0)),
            pl.BlockSpec((C_SC, N, L), lambda t: (0, 0, 0)),
            pl.BlockSpec((N_FLAG, N, L), lambda t: (0, 0, 0)),
            pl.BlockSpec((1, N_TFLAG, N, L), lambda t: (t, 0, 0, 0)),
            pl.BlockSpec((1, 6, N, L), lambda t: (t, 0, 0, 0)),
            smem((C_SF,), lambda t: (0,)),
            smem((C_SF,), lambda t: (0,)),
            smem((C_TF,), lambda t: (0,)),
            smem((C_TF,), lambda t: (0,)),
        ],
        out_specs=pl.BlockSpec((1, C_OUT, N, L), lambda t: (t, 0, 0, 0)),
        out_shape=jax.ShapeDtypeStruct((T, C_OUT, N, L), jnp.float32),
    )(inp_t, sc_rows, flag_t, tflag_t, temp_t, wd, w0, wtd, tw0)


def kernel(input_tensor, spa_cate, spa_flag, spa_temp_flag, spa_temp,
           W_cate, W_flag, W_tflag):
    wt = jnp.transpose(W_cate, (0, 2, 1))
    sct = jnp.transpose(spa_cate.astype(jnp.int32), (1, 0, 2))
    sc_rows = _sc_gather(wt, sct)

    inp_t = jnp.transpose(input_tensor, (1, 2, 0, 3))
    flag_t = jnp.transpose(spa_flag.astype(jnp.int32), (1, 0, 2))
    tflag_t = jnp.transpose(spa_temp_flag.astype(jnp.int32), (1, 2, 0, 3))
    temp_t = jnp.transpose(spa_temp, (1, 2, 0, 3))

    wd = (W_flag[:, 1, :] - W_flag[:, 0, :]).reshape(C_SF)
    w0 = W_flag[:, 0, :].reshape(C_SF)
    wtd = (W_tflag[:, 1, :] - W_tflag[:, 0, :]).reshape(C_TF)
    tw0 = W_tflag[:, 0, :].reshape(C_TF)

    res = _tc_assemble(inp_t, sc_rows, flag_t, tflag_t, temp_t,
                       wd, w0, wtd, tw0)
    return jnp.transpose(res, (2, 0, 1, 3))
